# baseline (speedup 1.0000x reference)
import functools
import math

import jax
import jax.numpy as jnp
from jax import lax
from jax.experimental import pallas as pl
from jax.experimental.pallas import tpu as pltpu
from jax.experimental.pallas import tpu_sc as plsc

_OUT_FEATURES = 100000
_K = 3
_SCALE = 32.0
_MARGIN = 0.2
_COS_M = math.cos(_MARGIN)
_SIN_M = math.sin(_MARGIN)
_TH = math.cos(math.pi - _MARGIN)
_MMM = 1.0 + math.cos(math.pi - _MARGIN)

_NW = 32
_B = 1024
_ROWS_PER_W = _B // _NW
_OUT_PER_W = _ROWS_PER_W * _OUT_FEATURES
_IN_PER_W = _OUT_PER_W * _K
_OUT_CH = 10000
_IN_CH = _OUT_CH * _K
_N_CH = _OUT_PER_W // _OUT_CH
_L = 16


def _sqrt(x):
    s = 0.5 * (1.0 + x)
    for _ in range(18):
        s = 0.5 * (s + x / s)
    return s


def _sc_body(cos_hbm, lbl_hbm, out_hbm, in_v, out_v, lblbuf_v, idx_v, g_v,
             pos_v, val_v, sem):
    wid = lax.axis_index("s") * 2 + lax.axis_index("c")
    in_base = wid * _IN_PER_W
    out_base = wid * _OUT_PER_W
    iota = lax.iota(jnp.int32, _L)
    iota3 = iota * _K

    pltpu.sync_copy(lbl_hbm.at[pl.ds(wid * _ROWS_PER_W, _ROWS_PER_W)],
                    lblbuf_v)
    for j in range(_ROWS_PER_W // _L):
        lbl = lblbuf_v[pl.ds(j * _L, _L)]
        row = wid * _ROWS_PER_W + j * _L + iota
        cin = row * (_OUT_FEATURES * _K) + lbl * _K
        for k in range(_K):
            idx_v[pl.ds(k * _ROWS_PER_W + j * _L, _L)] = cin + k
        pos_v[pl.ds(j * _L, _L)] = (j * _L + iota) * _OUT_FEATURES + lbl
    pltpu.async_copy(cos_hbm.at[idx_v], g_v, sem).wait()
    for j in range(_ROWS_PER_W // _L):
        c0 = g_v[pl.ds(j * _L, _L)]
        c1 = g_v[pl.ds(_ROWS_PER_W + j * _L, _L)]
        c2 = g_v[pl.ds(2 * _ROWS_PER_W + j * _L, _L)]
        c = jnp.maximum(jnp.maximum(c0, c1), c2)
        x = jnp.maximum(1.0 - c * c, 0.0)
        sine = _sqrt(x)
        phi = jnp.where(c > _TH, c * _COS_M - sine * _SIN_M, c - _MMM)
        val_v[pl.ds(j * _L, _L)] = _SCALE * phi

    def chunk_body(ch, _):
        pltpu.sync_copy(cos_hbm.at[pl.ds(in_base + ch * _IN_CH, _IN_CH)],
                        in_v)

        def vec_body(i, _):
            idx0 = i * (_L * _K) + iota3
            g0 = plsc.load_gather(in_v, [idx0])
            g1 = plsc.load_gather(in_v, [idx0 + 1])
            g2 = plsc.load_gather(in_v, [idx0 + 2])
            out_v[pl.ds(i * _L, _L)] = _SCALE * jnp.maximum(
                jnp.maximum(g0, g1), g2)
            return 0

        lax.fori_loop(0, _OUT_CH // _L, vec_body, 0)

        lo = ch * _OUT_CH
        for j in range(_ROWS_PER_W // _L):
            pos = pos_v[pl.ds(j * _L, _L)]
            m = (pos >= lo) & (pos < lo + _OUT_CH)
            plsc.store_scatter(out_v, [pos - lo],
                               val_v[pl.ds(j * _L, _L)], mask=m)

        pltpu.sync_copy(out_v,
                        out_hbm.at[pl.ds(out_base + ch * _OUT_CH, _OUT_CH)])
        return 0

    lax.fori_loop(0, _N_CH, chunk_body, 0)


@functools.partial(jax.jit, donate_argnums=())
def kernel(cosine, label):
    B = cosine.shape[0]
    cos_flat = cosine.reshape(-1)
    run = pl.kernel(
        _sc_body,
        out_type=jax.ShapeDtypeStruct((B * _OUT_FEATURES,), jnp.float32),
        mesh=plsc.VectorSubcoreMesh(core_axis_name="c", subcore_axis_name="s"),
        compiler_params=pltpu.CompilerParams(needs_layout_passes=False),
        scratch_types=[
            pltpu.VMEM((_IN_CH,), jnp.float32),
            pltpu.VMEM((_OUT_CH,), jnp.float32),
            pltpu.VMEM((_ROWS_PER_W,), jnp.int32),
            pltpu.VMEM((_K * _ROWS_PER_W,), jnp.int32),
            pltpu.VMEM((_K * _ROWS_PER_W,), jnp.float32),
            pltpu.VMEM((_ROWS_PER_W,), jnp.int32),
            pltpu.VMEM((_ROWS_PER_W,), jnp.float32),
            pltpu.SemaphoreType.DMA,
        ],
    )
    out = run(cos_flat, label)
    return out.reshape(B, _OUT_FEATURES)


# speedup vs baseline: 1.1000x; 1.1000x over previous
import functools
import math

import jax
import jax.numpy as jnp
from jax import lax
from jax.experimental import pallas as pl
from jax.experimental.pallas import tpu as pltpu
from jax.experimental.pallas import tpu_sc as plsc

_OUT_FEATURES = 100000
_K = 3
_SCALE = 32.0
_MARGIN = 0.2
_COS_M = math.cos(_MARGIN)
_SIN_M = math.sin(_MARGIN)
_TH = math.cos(math.pi - _MARGIN)
_MMM = 1.0 + math.cos(math.pi - _MARGIN)

_NW = 32
_B = 1024
_ROWS_PER_W = _B // _NW
_OUT_PER_W = _ROWS_PER_W * _OUT_FEATURES
_IN_PER_W = _OUT_PER_W * _K
_OUT_CH = 10000
_IN_CH = _OUT_CH * _K
_N_CH = _OUT_PER_W // _OUT_CH
_L = 16


def _sqrt(x):
    s = 0.5 * (1.0 + x)
    for _ in range(18):
        s = 0.5 * (s + x / s)
    return s


def _sc_body(cos_hbm, lbl_hbm, out_hbm, in_v0, in_v1, out_v0, out_v1,
             lblbuf_v, idx_v, g_v, pos_v, val_v, gsem,
             sem_in0, sem_in1, sem_out0, sem_out1):
    wid = lax.axis_index("s") * 2 + lax.axis_index("c")
    in_base = wid * _IN_PER_W
    out_base = wid * _OUT_PER_W
    iota = lax.iota(jnp.int32, _L)
    iota3 = iota * _K

    pltpu.sync_copy(lbl_hbm.at[pl.ds(wid * _ROWS_PER_W, _ROWS_PER_W)],
                    lblbuf_v)
    for j in range(_ROWS_PER_W // _L):
        lbl = lblbuf_v[pl.ds(j * _L, _L)]
        row = wid * _ROWS_PER_W + j * _L + iota
        cin = row * (_OUT_FEATURES * _K) + lbl * _K
        for k in range(_K):
            idx_v[pl.ds(k * _ROWS_PER_W + j * _L, _L)] = cin + k
        pos_v[pl.ds(j * _L, _L)] = (j * _L + iota) * _OUT_FEATURES + lbl
    pltpu.async_copy(cos_hbm.at[idx_v], g_v, gsem).wait()
    for j in range(_ROWS_PER_W // _L):
        c0 = g_v[pl.ds(j * _L, _L)]
        c1 = g_v[pl.ds(_ROWS_PER_W + j * _L, _L)]
        c2 = g_v[pl.ds(2 * _ROWS_PER_W + j * _L, _L)]
        c = jnp.maximum(jnp.maximum(c0, c1), c2)
        x = jnp.maximum(1.0 - c * c, 0.0)
        sine = _sqrt(x)
        phi = jnp.where(c > _TH, c * _COS_M - sine * _SIN_M, c - _MMM)
        val_v[pl.ds(j * _L, _L)] = _SCALE * phi

    in_bufs = (in_v0, in_v1)
    out_bufs = (out_v0, out_v1)
    sin = (sem_in0, sem_in1)
    sout = (sem_out0, sem_out1)

    def in_slice(ch):
        return cos_hbm.at[pl.ds(in_base + ch * _IN_CH, _IN_CH)]

    def out_slice(ch):
        return out_hbm.at[pl.ds(out_base + ch * _OUT_CH, _OUT_CH)]

    pltpu.async_copy(in_slice(0), in_v0, sem_in0)
    pltpu.async_copy(in_slice(1), in_v1, sem_in1)

    def outer(o, _):
        for b in range(2):
            ch = o * 2 + b
            ib, ob, si, so = in_bufs[b], out_bufs[b], sin[b], sout[b]
            pltpu.make_async_copy(in_slice(ch), ib, si).wait()

            @pl.when(ch >= 2)
            def _():
                pltpu.make_async_copy(ob, out_slice(ch - 2), so).wait()

            @functools.partial(plsc.parallel_loop, 0, _OUT_CH // _L,
                               unroll=8)
            def vec_body(i):
                idx0 = i * (_L * _K) + iota3
                g0 = plsc.load_gather(ib, [idx0])
                g1 = plsc.load_gather(ib, [idx0 + 1])
                g2 = plsc.load_gather(ib, [idx0 + 2])
                ob[pl.ds(i * _L, _L)] = _SCALE * jnp.maximum(
                    jnp.maximum(g0, g1), g2)

            @pl.when(ch + 2 < _N_CH)
            def _():
                pltpu.async_copy(in_slice(ch + 2), ib, si)

            lo = ch * _OUT_CH
            for j in range(_ROWS_PER_W // _L):
                pos = pos_v[pl.ds(j * _L, _L)]
                m = (pos >= lo) & (pos < lo + _OUT_CH)
                plsc.store_scatter(ob, [pos - lo],
                                   val_v[pl.ds(j * _L, _L)], mask=m)

            pltpu.async_copy(ob, out_slice(ch), so)
        return 0

    lax.fori_loop(0, _N_CH // 2, outer, 0)
    pltpu.make_async_copy(out_v0, out_slice(_N_CH - 2), sem_out0).wait()
    pltpu.make_async_copy(out_v1, out_slice(_N_CH - 1), sem_out1).wait()


@functools.partial(jax.jit, donate_argnums=())
def kernel(cosine, label):
    B = cosine.shape[0]
    cos_flat = cosine.reshape(-1)
    run = pl.kernel(
        _sc_body,
        out_type=jax.ShapeDtypeStruct((B * _OUT_FEATURES,), jnp.float32),
        mesh=plsc.VectorSubcoreMesh(core_axis_name="c", subcore_axis_name="s"),
        compiler_params=pltpu.CompilerParams(needs_layout_passes=False),
        scratch_types=[
            pltpu.VMEM((_IN_CH,), jnp.float32),
            pltpu.VMEM((_IN_CH,), jnp.float32),
            pltpu.VMEM((_OUT_CH,), jnp.float32),
            pltpu.VMEM((_OUT_CH,), jnp.float32),
            pltpu.VMEM((_ROWS_PER_W,), jnp.int32),
            pltpu.VMEM((_K * _ROWS_PER_W,), jnp.int32),
            pltpu.VMEM((_K * _ROWS_PER_W,), jnp.float32),
            pltpu.VMEM((_ROWS_PER_W,), jnp.int32),
            pltpu.VMEM((_ROWS_PER_W,), jnp.float32),
            pltpu.SemaphoreType.DMA,
            pltpu.SemaphoreType.DMA,
            pltpu.SemaphoreType.DMA,
            pltpu.SemaphoreType.DMA,
            pltpu.SemaphoreType.DMA,
        ],
    )
    out = run(cos_flat, label)
    return out.reshape(B, _OUT_FEATURES)


# speedup vs baseline: 6.4869x; 5.8972x over previous
import functools
import math

import jax
import jax.numpy as jnp
from jax import lax
from jax.experimental import pallas as pl
from jax.experimental.pallas import tpu as pltpu
from jax.experimental.pallas import tpu_sc as plsc

_OUT_FEATURES = 100000
_K = 3
_SCALE = 32.0
_INV_SCALE = 1.0 / 32.0
_MARGIN = 0.2
_COS_M = math.cos(_MARGIN)
_SIN_M = math.sin(_MARGIN)
_TH = math.cos(math.pi - _MARGIN)
_MMM = 1.0 + math.cos(math.pi - _MARGIN)

_NW = 32
_B = 1024
_ROWS_PER_W = _B // _NW
_L = 16

_OUT_FULL = 9984
_IN_FULL = _OUT_FULL * _K
_N_FULL = 10
_OUT_TAIL = 128
_IN_TAIL = _OUT_TAIL * _K
_TAIL_OUT0 = _N_FULL * _OUT_FULL
_TAIL_IN0 = _TAIL_OUT0 * _K
_NQ = _ROWS_PER_W * _N_FULL

_TC_COL0 = _TAIL_OUT0 + _OUT_TAIL
_TC_BR = 128
_TC_BW = 128


def _phi_patch(ob, lblbuf_v, iota, r_l, c_out0, n_out):
    ms, idxs = [], []
    for j in range(_ROWS_PER_W // _L):
        rows_j = iota + j * _L
        lbl = lblbuf_v[pl.ds(j * _L, _L)]
        m = (rows_j == r_l) & (lbl >= c_out0) & (lbl < c_out0 + n_out)
        idx = jnp.clip(lbl - c_out0, 0, n_out - 1)
        ms.append(m)
        idxs.append(idx)
    hit = jnp.any(ms[0] | ms[1])

    @pl.when(hit)
    def _():
        for m, idx in zip(ms, idxs):
            g = plsc.load_gather(ob, [idx], mask=m)
            c = g * _INV_SCALE
            x = jnp.maximum(1.0 - c * c, 0.0)
            s = 0.5 * (1.0 + x)
            for _ in range(12):
                s = 0.5 * (s + x / s)
            phi = jnp.where(c > _TH, c * _COS_M - s * _SIN_M, c - _MMM)
            plsc.store_scatter(ob, [idx], _SCALE * phi, mask=m)


def _sc_body(cos_hbm, lbl_hbm, out_hbm, in_v0, in_v1, out_v0, out_v1,
             lblbuf_v, lsem, sem_in0, sem_in1, sem_out0, sem_out1):
    wid = lax.axis_index("s") * 2 + lax.axis_index("c")
    row0 = wid * _ROWS_PER_W
    iota = lax.iota(jnp.int32, _L)
    iota3 = iota * _K

    pltpu.sync_copy(lbl_hbm.at[pl.ds(row0, _ROWS_PER_W)], lblbuf_v)

    in_bufs = (in_v0, in_v1)
    out_bufs = (out_v0, out_v1)
    sin = (sem_in0, sem_in1)
    sout = (sem_out0, sem_out1)

    def in_slice(r_l, sub):
        return cos_hbm.at[row0 + r_l, pl.ds(sub * _IN_FULL, _IN_FULL)]

    def out_slice(r_l, sub):
        return out_hbm.at[row0 + r_l, pl.ds(sub * _OUT_FULL, _OUT_FULL)]

    pltpu.async_copy(in_slice(0, 0), in_v0, sem_in0)
    pltpu.async_copy(in_slice(0, 1), in_v1, sem_in1)

    def outer(r, _):
        for sub in range(_N_FULL):
            b = sub % 2
            ib, ob, si, so = in_bufs[b], out_bufs[b], sin[b], sout[b]
            ch = r * _N_FULL + sub
            pltpu.make_async_copy(in_slice(r, sub), ib, si).wait()

            r_prev = r - (1 if sub < 2 else 0)
            sub_prev = (sub - 2) % _N_FULL

            @pl.when(ch >= 2)
            def _():
                pltpu.make_async_copy(ob, out_slice(r_prev, sub_prev),
                                      so).wait()

            def vec_body(ii, _):
                for u in range(8):
                    i = ii * 8 + u
                    idx0 = i * (_L * _K) + iota3
                    g0 = plsc.load_gather(ib, [idx0])
                    g1 = plsc.load_gather(ib, [idx0 + 1])
                    g2 = plsc.load_gather(ib, [idx0 + 2])
                    ob[pl.ds(i * _L, _L)] = _SCALE * jnp.maximum(
                        jnp.maximum(g0, g1), g2)
                return 0

            lax.fori_loop(0, _OUT_FULL // _L // 8, vec_body, 0)

            r_next = r + (1 if sub >= _N_FULL - 2 else 0)
            sub_next = (sub + 2) % _N_FULL

            @pl.when(ch + 2 < _NQ)
            def _():
                pltpu.async_copy(in_slice(r_next, sub_next), ib, si)

            _phi_patch(ob, lblbuf_v, iota, r, sub * _OUT_FULL, _OUT_FULL)
            pltpu.async_copy(ob, out_slice(r, sub), so)
        return 0

    lax.fori_loop(0, _ROWS_PER_W, outer, 0)
    pltpu.make_async_copy(out_v0, out_slice(_ROWS_PER_W - 1, _N_FULL - 2),
                          sem_out0).wait()
    pltpu.make_async_copy(out_v1, out_slice(_ROWS_PER_W - 1, _N_FULL - 1),
                          sem_out1).wait()

    def tin_slice(r_l):
        return cos_hbm.at[row0 + r_l, pl.ds(_TAIL_IN0, _IN_TAIL)]

    def tout_slice(r_l):
        return out_hbm.at[row0 + r_l, pl.ds(_TAIL_OUT0, _OUT_TAIL)]

    tin = (in_v0.at[pl.ds(0, _IN_TAIL)], in_v1.at[pl.ds(0, _IN_TAIL)])
    tout = (out_v0.at[pl.ds(0, _OUT_TAIL)], out_v1.at[pl.ds(0, _OUT_TAIL)])

    pltpu.async_copy(tin_slice(0), tin[0], sem_in0)
    pltpu.async_copy(tin_slice(1), tin[1], sem_in1)

    def touter(o, _):
        for b in range(2):
            r_l = o * 2 + b
            ib, ob, si, so = tin[b], tout[b], sin[b], sout[b]
            pltpu.make_async_copy(tin_slice(r_l), ib, si).wait()

            @pl.when(r_l >= 2)
            def _():
                pltpu.make_async_copy(ob, tout_slice(r_l - 2), so).wait()

            for i in range(_OUT_TAIL // _L):
                idx0 = i * (_L * _K) + iota3
                g0 = plsc.load_gather(ib, [idx0])
                g1 = plsc.load_gather(ib, [idx0 + 1])
                g2 = plsc.load_gather(ib, [idx0 + 2])
                ob[pl.ds(i * _L, _L)] = _SCALE * jnp.maximum(
                    jnp.maximum(g0, g1), g2)

            @pl.when(r_l + 2 < _ROWS_PER_W)
            def _():
                pltpu.async_copy(tin_slice(r_l + 2), ib, si)

            _phi_patch(ob, lblbuf_v, iota, r_l, _TAIL_OUT0, _OUT_TAIL)
            pltpu.async_copy(ob, tout_slice(r_l), so)
        return 0

    lax.fori_loop(0, _ROWS_PER_W // 2, touter, 0)
    pltpu.make_async_copy(tout[0], tout_slice(_ROWS_PER_W - 2),
                          sem_out0).wait()
    pltpu.make_async_copy(tout[1], tout_slice(_ROWS_PER_W - 1),
                          sem_out1).wait()


def _tc_tail_body(x_ref, lbl_ref, alias_ref, out_ref):
    del alias_ref
    x = x_ref[...]
    jdx = lax.broadcasted_iota(jnp.int32, (_TC_BW * _K, _TC_BW), 0)
    ldx = lax.broadcasted_iota(jnp.int32, (_TC_BW * _K, _TC_BW), 1)
    c = None
    for k in range(_K):
        sel = (jdx == _K * ldx + k).astype(jnp.float32)
        ck = lax.dot_general(x, sel, (((1,), (0,)), ((), ())),
                             precision=lax.Precision.HIGHEST,
                             preferred_element_type=jnp.float32)
        c = ck if c is None else jnp.maximum(c, ck)
    lbl = lbl_ref[...]
    cols = _TC_COL0 + lax.broadcasted_iota(jnp.int32, (_TC_BR, _TC_BW), 1)
    eq = cols == lbl
    c_l = jnp.sum(jnp.where(eq, c, 0.0), axis=1, keepdims=True)
    sine = jnp.sqrt(jnp.maximum(1.0 - c_l * c_l, 0.0))
    phi = jnp.where(c_l > _TH, c_l * _COS_M - sine * _SIN_M, c_l - _MMM)
    out_ref[...] = jnp.where(eq, _SCALE * phi, _SCALE * c)


def _tc_tail_fixup(cosine, label, sc_out):
    B = cosine.shape[0]
    return pl.pallas_call(
        _tc_tail_body,
        grid=(B // _TC_BR,),
        in_specs=[
            pl.BlockSpec((_TC_BR, _TC_BW * _K),
                         lambda i: (i, _TC_COL0 // _TC_BW)),
            pl.BlockSpec((_TC_BR, 1), lambda i: (i, 0)),
            pl.BlockSpec(memory_space=pl.ANY),
        ],
        out_specs=pl.BlockSpec((_TC_BR, _TC_BW),
                               lambda i: (i, _TC_COL0 // _TC_BW)),
        out_shape=jax.ShapeDtypeStruct((B, _OUT_FEATURES), jnp.float32),
        input_output_aliases={2: 0},
    )(cosine, label.reshape(B, 1), sc_out)


def kernel(cosine, label):
    B = cosine.shape[0]
    run = pl.kernel(
        _sc_body,
        out_type=jax.ShapeDtypeStruct((B, _OUT_FEATURES), jnp.float32),
        mesh=plsc.VectorSubcoreMesh(core_axis_name="c", subcore_axis_name="s"),
        compiler_params=pltpu.CompilerParams(needs_layout_passes=False),
        scratch_types=[
            pltpu.VMEM((_IN_FULL,), jnp.float32),
            pltpu.VMEM((_IN_FULL,), jnp.float32),
            pltpu.VMEM((_OUT_FULL,), jnp.float32),
            pltpu.VMEM((_OUT_FULL,), jnp.float32),
            pltpu.VMEM((_ROWS_PER_W,), jnp.int32),
            pltpu.SemaphoreType.DMA,
            pltpu.SemaphoreType.DMA,
            pltpu.SemaphoreType.DMA,
            pltpu.SemaphoreType.DMA,
            pltpu.SemaphoreType.DMA,
        ],
    )
    sc_out = run(cosine, label)
    return _tc_tail_fixup(cosine, label, sc_out)


# speedup vs baseline: 9.0568x; 1.3962x over previous
import functools
import math

import jax
import jax.numpy as jnp
from jax import lax
from jax.experimental import pallas as pl
from jax.experimental.pallas import tpu as pltpu
from jax.experimental.pallas import tpu_sc as plsc

_OUT_FEATURES = 100000
_K = 3
_SCALE = 32.0
_INV_SCALE = 1.0 / 32.0
_MARGIN = 0.2
_COS_M = math.cos(_MARGIN)
_SIN_M = math.sin(_MARGIN)
_TH = math.cos(math.pi - _MARGIN)
_MMM = 1.0 + math.cos(math.pi - _MARGIN)

_NW = 32
_B = 1024
_ROWS_PER_W = _B // _NW
_L = 16

_OUT_FULL = 9984
_IN_FULL = _OUT_FULL * _K
_N_FULL = 10
_OUT_TAIL = 128
_IN_TAIL = _OUT_TAIL * _K
_TAIL_OUT0 = _N_FULL * _OUT_FULL
_TAIL_IN0 = _TAIL_OUT0 * _K
_NQ = _ROWS_PER_W * _N_FULL

_TC_COL0 = _TAIL_OUT0 + _OUT_TAIL
_TC_BR = 128
_TC_BW = 128


def _phi_patch(ob, lblbuf_v, iota, r_l, c_out0, n_out):
    ms, idxs = [], []
    for j in range(_ROWS_PER_W // _L):
        rows_j = iota + j * _L
        lbl = lblbuf_v[pl.ds(j * _L, _L)]
        m = (rows_j == r_l) & (lbl >= c_out0) & (lbl < c_out0 + n_out)
        idx = jnp.clip(lbl - c_out0, 0, n_out - 1)
        ms.append(m)
        idxs.append(idx)
    hit = jnp.any(ms[0] | ms[1])

    @pl.when(hit)
    def _():
        for m, idx in zip(ms, idxs):
            g = plsc.load_gather(ob, [idx], mask=m)
            c = g * _INV_SCALE
            x = jnp.maximum(1.0 - c * c, 0.0)
            s = 0.5 * (1.0 + x)
            for _ in range(12):
                s = 0.5 * (s + x / s)
            phi = jnp.where(c > _TH, c * _COS_M - s * _SIN_M, c - _MMM)
            plsc.store_scatter(ob, [idx], _SCALE * phi, mask=m)


def _sc_body(cos_hbm, lbl_hbm, out_hbm, in_v0, in_v1, out_v0, out_v1,
             lblbuf_v, lsem, sem_in0, sem_in1, sem_out0, sem_out1):
    wid = lax.axis_index("s") * 2 + lax.axis_index("c")
    row0 = wid * _ROWS_PER_W
    iota = lax.iota(jnp.int32, _L)
    iota3 = iota * _K

    pltpu.sync_copy(lbl_hbm.at[pl.ds(row0, _ROWS_PER_W)], lblbuf_v)

    in_bufs = (in_v0, in_v1)
    out_bufs = (out_v0, out_v1)
    sin = (sem_in0, sem_in1)
    sout = (sem_out0, sem_out1)

    def in_slice(r_l, sub):
        return cos_hbm.at[row0 + r_l, pl.ds(sub * _IN_FULL, _IN_FULL)]

    def out_slice(r_l, sub):
        return out_hbm.at[row0 + r_l, pl.ds(sub * _OUT_FULL, _OUT_FULL)]

    pltpu.async_copy(in_slice(0, 0), in_v0, sem_in0)
    pltpu.async_copy(in_slice(0, 1), in_v1, sem_in1)

    def outer(r, _):
        for sub in range(_N_FULL):
            b = sub % 2
            ib, ob, si, so = in_bufs[b], out_bufs[b], sin[b], sout[b]
            ch = r * _N_FULL + sub
            pltpu.make_async_copy(in_slice(r, sub), ib, si).wait()

            r_prev = r - (1 if sub < 2 else 0)
            sub_prev = (sub - 2) % _N_FULL

            @pl.when(ch >= 2)
            def _():
                pltpu.make_async_copy(ob, out_slice(r_prev, sub_prev),
                                      so).wait()

            def vec_body(ii, _):
                gs = []
                for u in range(16):
                    i = ii * 16 + u
                    idx0 = i * (_L * _K) + iota3
                    gs.append((i, plsc.load_gather(ib, [idx0]),
                               plsc.load_gather(ib, [idx0 + 1]),
                               plsc.load_gather(ib, [idx0 + 2])))
                for i, g0, g1, g2 in gs:
                    ob[pl.ds(i * _L, _L)] = _SCALE * jnp.maximum(
                        jnp.maximum(g0, g1), g2)
                return 0

            lax.fori_loop(0, _OUT_FULL // _L // 16, vec_body, 0)

            r_next = r + (1 if sub >= _N_FULL - 2 else 0)
            sub_next = (sub + 2) % _N_FULL

            @pl.when(ch + 2 < _NQ)
            def _():
                pltpu.async_copy(in_slice(r_next, sub_next), ib, si)

            _phi_patch(ob, lblbuf_v, iota, r, sub * _OUT_FULL, _OUT_FULL)
            pltpu.async_copy(ob, out_slice(r, sub), so)
        return 0

    lax.fori_loop(0, _ROWS_PER_W, outer, 0)
    pltpu.make_async_copy(out_v0, out_slice(_ROWS_PER_W - 1, _N_FULL - 2),
                          sem_out0).wait()
    pltpu.make_async_copy(out_v1, out_slice(_ROWS_PER_W - 1, _N_FULL - 1),
                          sem_out1).wait()

    def tin_slice(r_l):
        return cos_hbm.at[row0 + r_l, pl.ds(_TAIL_IN0, _IN_TAIL)]

    def tout_slice(r_l):
        return out_hbm.at[row0 + r_l, pl.ds(_TAIL_OUT0, _OUT_TAIL)]

    tin = (in_v0.at[pl.ds(0, _IN_TAIL)], in_v1.at[pl.ds(0, _IN_TAIL)])
    tout = (out_v0.at[pl.ds(0, _OUT_TAIL)], out_v1.at[pl.ds(0, _OUT_TAIL)])

    pltpu.async_copy(tin_slice(0), tin[0], sem_in0)
    pltpu.async_copy(tin_slice(1), tin[1], sem_in1)

    def touter(o, _):
        for b in range(2):
            r_l = o * 2 + b
            ib, ob, si, so = tin[b], tout[b], sin[b], sout[b]
            pltpu.make_async_copy(tin_slice(r_l), ib, si).wait()

            @pl.when(r_l >= 2)
            def _():
                pltpu.make_async_copy(ob, tout_slice(r_l - 2), so).wait()

            for i in range(_OUT_TAIL // _L):
                idx0 = i * (_L * _K) + iota3
                g0 = plsc.load_gather(ib, [idx0])
                g1 = plsc.load_gather(ib, [idx0 + 1])
                g2 = plsc.load_gather(ib, [idx0 + 2])
                ob[pl.ds(i * _L, _L)] = _SCALE * jnp.maximum(
                    jnp.maximum(g0, g1), g2)

            @pl.when(r_l + 2 < _ROWS_PER_W)
            def _():
                pltpu.async_copy(tin_slice(r_l + 2), ib, si)

            _phi_patch(ob, lblbuf_v, iota, r_l, _TAIL_OUT0, _OUT_TAIL)
            pltpu.async_copy(ob, tout_slice(r_l), so)
        return 0

    lax.fori_loop(0, _ROWS_PER_W // 2, touter, 0)
    pltpu.make_async_copy(tout[0], tout_slice(_ROWS_PER_W - 2),
                          sem_out0).wait()
    pltpu.make_async_copy(tout[1], tout_slice(_ROWS_PER_W - 1),
                          sem_out1).wait()


def _tc_tail_body(x_ref, lbl_ref, alias_ref, out_ref):
    del alias_ref
    x = x_ref[...]
    jdx = lax.broadcasted_iota(jnp.int32, (_TC_BW * _K, _TC_BW), 0)
    ldx = lax.broadcasted_iota(jnp.int32, (_TC_BW * _K, _TC_BW), 1)
    c = None
    for k in range(_K):
        sel = (jdx == _K * ldx + k).astype(jnp.float32)
        ck = lax.dot_general(x, sel, (((1,), (0,)), ((), ())),
                             precision=lax.Precision.HIGHEST,
                             preferred_element_type=jnp.float32)
        c = ck if c is None else jnp.maximum(c, ck)
    lbl = lbl_ref[...]
    cols = _TC_COL0 + lax.broadcasted_iota(jnp.int32, (_TC_BR, _TC_BW), 1)
    eq = cols == lbl
    c_l = jnp.sum(jnp.where(eq, c, 0.0), axis=1, keepdims=True)
    sine = jnp.sqrt(jnp.maximum(1.0 - c_l * c_l, 0.0))
    phi = jnp.where(c_l > _TH, c_l * _COS_M - sine * _SIN_M, c_l - _MMM)
    out_ref[...] = jnp.where(eq, _SCALE * phi, _SCALE * c)


def _tc_tail_fixup(cosine, label, sc_out):
    B = cosine.shape[0]
    return pl.pallas_call(
        _tc_tail_body,
        grid=(B // _TC_BR,),
        in_specs=[
            pl.BlockSpec((_TC_BR, _TC_BW * _K),
                         lambda i: (i, _TC_COL0 // _TC_BW)),
            pl.BlockSpec((_TC_BR, 1), lambda i: (i, 0)),
            pl.BlockSpec(memory_space=pl.ANY),
        ],
        out_specs=pl.BlockSpec((_TC_BR, _TC_BW),
                               lambda i: (i, _TC_COL0 // _TC_BW)),
        out_shape=jax.ShapeDtypeStruct((B, _OUT_FEATURES), jnp.float32),
        input_output_aliases={2: 0},
    )(cosine, label.reshape(B, 1), sc_out)


def kernel(cosine, label):
    B = cosine.shape[0]
    run = pl.kernel(
        _sc_body,
        out_type=jax.ShapeDtypeStruct((B, _OUT_FEATURES), jnp.float32),
        mesh=plsc.VectorSubcoreMesh(core_axis_name="c", subcore_axis_name="s"),
        compiler_params=pltpu.CompilerParams(needs_layout_passes=False),
        scratch_types=[
            pltpu.VMEM((_IN_FULL,), jnp.float32),
            pltpu.VMEM((_IN_FULL,), jnp.float32),
            pltpu.VMEM((_OUT_FULL,), jnp.float32),
            pltpu.VMEM((_OUT_FULL,), jnp.float32),
            pltpu.VMEM((_ROWS_PER_W,), jnp.int32),
            pltpu.SemaphoreType.DMA,
            pltpu.SemaphoreType.DMA,
            pltpu.SemaphoreType.DMA,
            pltpu.SemaphoreType.DMA,
            pltpu.SemaphoreType.DMA,
        ],
    )
    sc_out = run(cosine, label)
    return _tc_tail_fixup(cosine, label, sc_out)
